# baseline (speedup 1.0000x reference)
import functools

import jax
import jax.numpy as jnp
from jax.experimental import pallas as pl
from jax.experimental.pallas import tpu as pltpu

_ROWS = 2


def _weights_all(c0, c1, lengths, L):
    B = c0.shape[0]
    idx = jax.lax.broadcasted_iota(jnp.int32, (B, L), 1)
    valid = idx < lengths
    same = (c0 == pltpu.roll(c0, 1, axis=1)) & (c1 == pltpu.roll(c1, 1, axis=1))
    ng = ((idx == 0) | jnp.logical_not(same)) & valid

    s = jnp.where(ng, idx, -1)
    k = 1
    while k < L:
        s = jnp.maximum(s, jnp.where(idx >= k, pltpu.roll(s, k, axis=1), -1))
        k *= 2
    t = jnp.where(ng, idx, L)
    t = jnp.where(idx < L - 1, pltpu.roll(t, L - 1, axis=1), L)
    k = 1
    while k < L:
        t = jnp.minimum(t, jnp.where(idx < L - k, pltpu.roll(t, L - k, axis=1), L))
        k *= 2

    run_len = (jnp.minimum(t, lengths) - s).astype(jnp.float32)
    num_groups = jnp.sum(ng.astype(jnp.float32), axis=1, keepdims=True)
    denom = num_groups * run_len
    safe = valid & (denom > 0.0)
    return jnp.where(safe, 1.0 / jnp.where(safe, denom, 1.0), 0.0)


def _pool_kernel(len_ref, vq_ref, feat_ref, out_ref, w_ref, *, B, L):
    g = pl.program_id(0)

    @pl.when(g == 0)
    def _():
        c0 = vq_ref[:, 0, :]
        c1 = vq_ref[:, 1, :]
        lengths = jnp.concatenate(
            [jnp.full((1, 1), len_ref[i], jnp.int32) for i in range(B)], axis=0)
        w_ref[...] = _weights_all(c0, c1, lengths, L)

    for r in range(_ROWS):
        w_row = w_ref[pl.ds(g * _ROWS + r, 1), :]
        out_ref[r] = jnp.dot(w_row, feat_ref[r, 0],
                             preferred_element_type=jnp.float32)


@jax.jit
def kernel(input_feature, input_lengths, vq_indices):
    B, N, L, D = input_feature.shape
    lengths = input_lengths.astype(jnp.int32)
    vq_t = jnp.transpose(vq_indices.astype(jnp.int32), (0, 2, 1))

    grid_spec = pltpu.PrefetchScalarGridSpec(
        num_scalar_prefetch=1,
        grid=(B // _ROWS,),
        in_specs=[
            pl.BlockSpec((B, 2, L), lambda g, lens: (0, 0, 0)),
            pl.BlockSpec((_ROWS, 1, L, D), lambda g, lens: (g, N - 1, 0, 0)),
        ],
        out_specs=pl.BlockSpec((_ROWS, 1, D), lambda g, lens: (g, 0, 0)),
        scratch_shapes=[pltpu.VMEM((B, L), jnp.float32)],
    )
    out = pl.pallas_call(
        functools.partial(_pool_kernel, B=B, L=L),
        grid_spec=grid_spec,
        out_shape=jax.ShapeDtypeStruct((B, 1, D), jnp.float32),
    )(lengths, vq_t, input_feature)
    return out[:, 0, :]
